# Initial kernel scaffold; baseline (speedup 1.0000x reference)
import jax
import jax.numpy as jnp
from jax.experimental import pallas as pl


def kernel(x, edge_index_0, edge_index_1, root_users, params):
    raise NotImplementedError("write your pallas kernel here")


# speedup vs baseline: 6.4291x; 6.4291x over previous
import functools

import jax
import jax.numpy as jnp
from jax import lax
from jax.experimental import pallas as pl
from jax.experimental.pallas import tpu as pltpu
from jax.experimental.pallas import tpu_sc as plsc

N = 10000
E = 320000
D = 128
NI = 2
EPS = 1e-5

NC = 2
NS = 16
CH = 80
ZR = 104

_MESH = dict(core_axis_name="c", subcore_axis_name="s")


def _zero_fill(z_v, rows, width):
    z16 = jnp.zeros((16,), jnp.float32)

    def body(r, _):
        for k in range(width // 16):
            z_v[r, pl.ds(k * 16, 16)] = z16
        return 0

    lax.fori_loop(0, rows, body, 0)


def _seg_sum(table, src_idx, dst_idx):
    epw = E // NS
    nsteps = epw // CH
    rows_pw = 624
    tail = N - NS * rows_pw

    @functools.partial(
        pl.kernel,
        out_type=jax.ShapeDtypeStruct((NC, N, D), jnp.float32),
        mesh=plsc.VectorSubcoreMesh(**_MESH),
        scratch_types=[
            pltpu.VMEM_SHARED((N, D), jnp.float32),
            pltpu.VMEM((CH,), jnp.int32),
            pltpu.VMEM((CH,), jnp.int32),
            pltpu.VMEM((CH, D), jnp.float32),
            pltpu.VMEM((ZR, D), jnp.float32),
            pltpu.SemaphoreType.DMA,
        ],
    )
    def k(table_hbm, src_hbm, dst_hbm, out_hbm, acc, src_v, dst_v, rows_v,
          z_v, sem):
        c = lax.axis_index("c")
        s = lax.axis_index("s")
        _zero_fill(z_v, ZR, D)
        for q in range(rows_pw // ZR):
            pltpu.sync_copy(z_v, acc.at[pl.ds(s * rows_pw + q * ZR, ZR)])

        @pl.when(s == 0)
        def _():
            pltpu.sync_copy(z_v.at[pl.ds(0, tail)],
                            acc.at[pl.ds(NS * rows_pw, tail)])

        plsc.subcore_barrier()

        def step(j, _):
            base = c * E + s * epw + j * CH
            pltpu.sync_copy(src_hbm.at[pl.ds(base, CH)], src_v)
            pltpu.sync_copy(dst_hbm.at[pl.ds(base, CH)], dst_v)
            pltpu.async_copy(table_hbm.at[src_v], rows_v, sem).wait()
            pltpu.sync_copy(rows_v, acc.at[dst_v], add=True)
            return 0

        lax.fori_loop(0, nsteps, step, 0)
        plsc.subcore_barrier()
        pltpu.sync_copy(acc.at[pl.ds(s * rows_pw, rows_pw)],
                        out_hbm.at[c, pl.ds(s * rows_pw, rows_pw)])

        @pl.when(s == 0)
        def _():
            pltpu.sync_copy(acc.at[pl.ds(NS * rows_pw, tail)],
                            out_hbm.at[c, pl.ds(NS * rows_pw, tail)])

    return k(table, src_idx, dst_idx)


def _counts(dst_idx):
    epw = E // NS
    nsteps = epw // CH
    rows_pw = 624
    tail = N - NS * rows_pw

    @functools.partial(
        pl.kernel,
        out_type=jax.ShapeDtypeStruct((NC, N, D), jnp.float32),
        mesh=plsc.VectorSubcoreMesh(**_MESH),
        scratch_types=[
            pltpu.VMEM_SHARED((N, D), jnp.float32),
            pltpu.VMEM((CH,), jnp.int32),
            pltpu.VMEM((CH, D), jnp.float32),
            pltpu.VMEM((ZR, D), jnp.float32),
        ],
    )
    def k(dst_hbm, out_hbm, acc, dst_v, ones_v, z_v):
        c = lax.axis_index("c")
        s = lax.axis_index("s")
        _zero_fill(z_v, ZR, D)
        one16 = jnp.ones((16,), jnp.float32)

        def fill_ones(r, _):
            for kk in range(D // 16):
                ones_v[r, pl.ds(kk * 16, 16)] = one16
            return 0

        lax.fori_loop(0, CH, fill_ones, 0)
        for q in range(rows_pw // ZR):
            pltpu.sync_copy(z_v, acc.at[pl.ds(s * rows_pw + q * ZR, ZR)])

        @pl.when(s == 0)
        def _():
            pltpu.sync_copy(z_v.at[pl.ds(0, tail)],
                            acc.at[pl.ds(NS * rows_pw, tail)])

        plsc.subcore_barrier()

        def step(j, _):
            base = c * E + s * epw + j * CH
            pltpu.sync_copy(dst_hbm.at[pl.ds(base, CH)], dst_v)
            pltpu.sync_copy(ones_v, acc.at[dst_v], add=True)
            return 0

        lax.fori_loop(0, nsteps, step, 0)
        plsc.subcore_barrier()
        pltpu.sync_copy(acc.at[pl.ds(s * rows_pw, rows_pw)],
                        out_hbm.at[c, pl.ds(s * rows_pw, rows_pw)])

        @pl.when(s == 0)
        def _():
            pltpu.sync_copy(acc.at[pl.ds(NS * rows_pw, tail)],
                            out_hbm.at[c, pl.ds(NS * rows_pw, tail)])

    return k(dst_idx)


def _gather_rows(table, idx):
    rows_pw = 400
    nsteps = rows_pw // CH

    @functools.partial(
        pl.kernel,
        out_type=jax.ShapeDtypeStruct((N, D), jnp.float32),
        mesh=plsc.VectorSubcoreMesh(**_MESH),
        scratch_types=[
            pltpu.VMEM((CH,), jnp.int32),
            pltpu.VMEM((CH, D), jnp.float32),
            pltpu.SemaphoreType.DMA,
        ],
    )
    def k(table_hbm, idx_hbm, out_hbm, idx_v, rows_v, sem):
        c = lax.axis_index("c")
        s = lax.axis_index("s")
        wid = c * NS + s

        @pl.when(wid < N // rows_pw)
        def _():
            def step(j, _):
                base = wid * rows_pw + j * CH
                pltpu.sync_copy(idx_hbm.at[pl.ds(base, CH)], idx_v)
                pltpu.async_copy(table_hbm.at[idx_v], rows_v, sem).wait()
                pltpu.sync_copy(rows_v, out_hbm.at[pl.ds(base, CH)])
                return 0

            lax.fori_loop(0, nsteps, step, 0)

    return k(table, idx)


BR = 1000


def _xw_body(x_ref, w_ref, o_ref):
    o_ref[...] = jnp.dot(x_ref[0], w_ref[0],
                         preferred_element_type=jnp.float32)[None]


def _tc_xw(xs, W):
    return pl.pallas_call(
        _xw_body,
        grid=(2, N // BR),
        in_specs=[
            pl.BlockSpec((1, BR, D), lambda a, b: (a, b, 0)),
            pl.BlockSpec((1, D, D), lambda a, b: (a, 0, 0)),
        ],
        out_specs=pl.BlockSpec((1, BR, D), lambda a, b: (a, b, 0)),
        out_shape=jax.ShapeDtypeStruct((2, N, D), jnp.float32),
    )(xs, W)


def _scale_body(raw_ref, cnt_ref, o_ref):
    cvec = cnt_ref[0][:, 0:1]
    inv = jnp.where(cvec > 0, 1.0 / cvec, 0.0)
    o_ref[...] = (raw_ref[0] * inv)[None]


def _tc_scale(raw, counts4, i):
    return pl.pallas_call(
        _scale_body,
        grid=(2, N // BR),
        in_specs=[
            pl.BlockSpec((1, BR, D), lambda a, b: (a, b, 0)),
            pl.BlockSpec((1, BR, D), lambda a, b: (2 * i + 1 - a, b, 0)),
        ],
        out_specs=pl.BlockSpec((1, BR, D), lambda a, b: (a, b, 0)),
        out_shape=jax.ShapeDtypeStruct((2, N, D), jnp.float32),
    )(raw, counts4)


def _post_body(raw_ref, cnt_ref, st_ref, nz_ref, bhg_ref, wm_ref, bm_ref,
               ws_ref, bs_ref, g_ref, b_ref, o_ref):
    cvec = cnt_ref[0][:, 0:1]
    dinv = jnp.where(cvec > 0, 1.0 / cvec, 0.0)
    conv = raw_ref[0] * dinv + bhg_ref[0]
    h = jnp.maximum(conv, 0.0)
    mean = jnp.dot(h, wm_ref[0], preferred_element_type=jnp.float32) \
        + bm_ref[0]
    logstd = jnp.dot(h, ws_ref[0], preferred_element_type=jnp.float32) \
        + bs_ref[0]
    enc = nz_ref[0] * jnp.exp(logstd) + mean
    y = enc + st_ref[0]
    mu = jnp.mean(y, axis=-1, keepdims=True)
    var = jnp.mean((y - mu) ** 2, axis=-1, keepdims=True)
    o_ref[...] = ((y - mu) / jnp.sqrt(var + EPS) * g_ref[0] + b_ref[0])[None]


def _tc_post(raw, counts4, i, state, noise, bhg, Wm, bm, Ws, bs, g, b):
    row = lambda a, bb: (a, bb, 0)
    par = lambda a, bb: (a, 0, 0)
    return pl.pallas_call(
        _post_body,
        grid=(2, N // BR),
        in_specs=[
            pl.BlockSpec((1, BR, D), row),
            pl.BlockSpec((1, BR, D), lambda a, bb: (2 * i + a, bb, 0)),
            pl.BlockSpec((1, BR, D), row),
            pl.BlockSpec((1, BR, D), row),
            pl.BlockSpec((1, 1, D), par),
            pl.BlockSpec((1, D, D), par),
            pl.BlockSpec((1, 1, D), par),
            pl.BlockSpec((1, D, D), par),
            pl.BlockSpec((1, 1, D), par),
            pl.BlockSpec((1, 1, D), par),
            pl.BlockSpec((1, 1, D), par),
        ],
        out_specs=pl.BlockSpec((1, BR, D), row),
        out_shape=jax.ShapeDtypeStruct((2, N, D), jnp.float32),
    )(raw, counts4, state, noise, bhg, Wm, bm, Ws, bs, g, b)


def kernel(x, edge_index_0, edge_index_1, root_users, params):
    graphs = [edge_index_0.astype(jnp.int32), edge_index_1.astype(jnp.int32)]
    root = root_users.astype(jnp.int32)

    key = jax.random.key(42)
    noises = [jax.random.normal(jax.random.fold_in(key, t), (N, D),
                                jnp.float32) for t in range(2 * NI)]

    cnt_r0 = _counts(jnp.concatenate([graphs[0][0], graphs[1][0]]))
    cnt_r1 = _counts(jnp.concatenate([graphs[0][1], graphs[1][1]]))
    counts4 = jnp.stack([cnt_r0[0], cnt_r1[0], cnt_r0[1], cnt_r1[1]])

    cas0 = _gather_rows(x, root)
    state = jnp.stack([x, cas0])

    pu, pc = params['user_vae'], params['cas_vae']
    Whg = jnp.stack([pu['Whg'], pc['Whg']])
    Wm = jnp.stack([pu['Wm'], pc['Wm']])
    Ws = jnp.stack([pu['Ws'], pc['Ws']])
    bhg = jnp.stack([pu['bhg'], pc['bhg']])[:, None, :]
    bm = jnp.stack([pu['bm'], pc['bm']])[:, None, :]
    bs = jnp.stack([pu['bs'], pc['bs']])[:, None, :]

    outs = []
    for i in range(NI):
        g0, g1 = graphs[i][0], graphs[i][1]
        src_p1 = jnp.concatenate([g0, g1 + N])
        dst_p1 = jnp.concatenate([g1, g0])
        src_p2 = jnp.concatenate([g1, g0 + N])
        dst_p2 = jnp.concatenate([g0, g1])

        xw = _tc_xw(state, Whg)
        raw1 = _seg_sum(xw.reshape(2 * N, D), src_p1, dst_p1)
        efeat = _tc_scale(raw1, counts4, i)
        raw2 = _seg_sum(efeat.reshape(2 * N, D), src_p2, dst_p2)

        noise_i = jnp.stack([noises[2 * i], noises[2 * i + 1]])
        lng = jnp.stack([params['un_g'][i], params['cn_g'][i]])[:, None, :]
        lnb = jnp.stack([params['un_b'][i], params['cn_b'][i]])[:, None, :]
        state = _tc_post(raw2, counts4, i, state, noise_i, bhg, Wm, bm, Ws,
                         bs, lng, lnb)
        outs.append(state[0])
        outs.append(state[1])
    return tuple(outs)


# speedup vs baseline: 10.3565x; 1.6109x over previous
import functools

import jax
import jax.numpy as jnp
from jax import lax
from jax.experimental import pallas as pl
from jax.experimental.pallas import tpu as pltpu
from jax.experimental.pallas import tpu_sc as plsc

N = 10000
E = 320000
D = 128
NI = 2
EPS = 1e-5

NC = 2
NS = 16
CH = 40
ZR = 104

_MESH = dict(core_axis_name="c", subcore_axis_name="s")


def _zero_fill(z_v, rows, width):
    z16 = jnp.zeros((16,), jnp.float32)

    def body(r, _):
        for k in range(width // 16):
            z_v[r, pl.ds(k * 16, 16)] = z16
        return 0

    lax.fori_loop(0, rows, body, 0)


RB = 5


def _seg_sum(table, src_idx, dst_idx):
    epw = E // NS
    nblk = epw // (RB * CH)
    rows_pw = 624
    tail = N - NS * rows_pw
    src3 = src_idx.reshape(-1, RB, CH)
    dst3 = dst_idx.reshape(-1, RB, CH)

    @functools.partial(
        pl.kernel,
        out_type=jax.ShapeDtypeStruct((NC, N, D), jnp.float32),
        mesh=plsc.VectorSubcoreMesh(**_MESH),
        scratch_types=[
            pltpu.VMEM_SHARED((N, D), jnp.float32),
            pltpu.VMEM((RB, CH), jnp.int32),
            pltpu.VMEM((RB, CH), jnp.int32),
            [pltpu.VMEM((CH, D), jnp.float32) for _ in range(RB)],
            pltpu.VMEM((ZR, D), jnp.float32),
            [pltpu.SemaphoreType.DMA for _ in range(RB)],
            [pltpu.SemaphoreType.DMA for _ in range(RB)],
        ],
    )
    def k(table_hbm, src_hbm, dst_hbm, out_hbm, acc, src_v, dst_v, bufs,
          z_v, gsems, ssems):
        c = lax.axis_index("c")
        s = lax.axis_index("s")
        _zero_fill(z_v, ZR, D)
        for q in range(rows_pw // ZR):
            pltpu.sync_copy(z_v, acc.at[pl.ds(s * rows_pw + q * ZR, ZR)])

        @pl.when(s == 0)
        def _():
            pltpu.sync_copy(z_v.at[pl.ds(0, tail)],
                            acc.at[pl.ds(NS * rows_pw, tail)])

        plsc.subcore_barrier()

        def step(g, _):
            blk = (c * NS + s) * nblk + g
            pltpu.sync_copy(src_hbm.at[blk], src_v)
            pltpu.sync_copy(dst_hbm.at[blk], dst_v)

            @pl.when(g > 0)
            def _():
                for r in range(RB):
                    pltpu.make_async_copy(bufs[r], acc.at[dst_v.at[r]],
                                          ssems[r]).wait()
            gs = [pltpu.async_copy(table_hbm.at[src_v.at[r]], bufs[r],
                                   gsems[r]) for r in range(RB)]
            for r in range(RB):
                gs[r].wait()
                pltpu.async_copy(bufs[r], acc.at[dst_v.at[r]], ssems[r],
                                 add=True)
            return 0

        lax.fori_loop(0, nblk, step, 0)
        for r in range(RB):
            pltpu.make_async_copy(bufs[r], acc.at[dst_v.at[r]],
                                  ssems[r]).wait()
        plsc.subcore_barrier()
        pltpu.sync_copy(acc.at[pl.ds(s * rows_pw, rows_pw)],
                        out_hbm.at[c, pl.ds(s * rows_pw, rows_pw)])

        @pl.when(s == 0)
        def _():
            pltpu.sync_copy(acc.at[pl.ds(NS * rows_pw, tail)],
                            out_hbm.at[c, pl.ds(NS * rows_pw, tail)])

    return k(table, src3, dst3)


def _counts(dst_idx):
    epw = E // NS
    nblk = epw // (RB * CH)
    rows_pw = 624
    tail = N - NS * rows_pw
    dst3 = dst_idx.reshape(-1, RB, CH)

    @functools.partial(
        pl.kernel,
        out_type=jax.ShapeDtypeStruct((NC, N, D), jnp.float32),
        mesh=plsc.VectorSubcoreMesh(**_MESH),
        scratch_types=[
            pltpu.VMEM_SHARED((N, D), jnp.float32),
            pltpu.VMEM((RB, CH), jnp.int32),
            pltpu.VMEM((CH, D), jnp.float32),
            pltpu.VMEM((ZR, D), jnp.float32),
            pltpu.SemaphoreType.DMA,
        ],
    )
    def k(dst_hbm, out_hbm, acc, dst_v, ones_v, z_v, ssem):
        c = lax.axis_index("c")
        s = lax.axis_index("s")
        _zero_fill(z_v, ZR, D)
        one16 = jnp.ones((16,), jnp.float32)

        def fill_ones(r, _):
            for kk in range(D // 16):
                ones_v[r, pl.ds(kk * 16, 16)] = one16
            return 0

        lax.fori_loop(0, CH, fill_ones, 0)
        for q in range(rows_pw // ZR):
            pltpu.sync_copy(z_v, acc.at[pl.ds(s * rows_pw + q * ZR, ZR)])

        @pl.when(s == 0)
        def _():
            pltpu.sync_copy(z_v.at[pl.ds(0, tail)],
                            acc.at[pl.ds(NS * rows_pw, tail)])

        plsc.subcore_barrier()

        def step(g, _):
            blk = (c * NS + s) * nblk + g
            pltpu.sync_copy(dst_hbm.at[blk], dst_v)
            ss = [pltpu.async_copy(ones_v, acc.at[dst_v.at[r]], ssem,
                                   add=True) for r in range(RB)]
            for r in range(RB):
                ss[r].wait()
            return 0

        lax.fori_loop(0, nblk, step, 0)
        plsc.subcore_barrier()
        pltpu.sync_copy(acc.at[pl.ds(s * rows_pw, rows_pw)],
                        out_hbm.at[c, pl.ds(s * rows_pw, rows_pw)])

        @pl.when(s == 0)
        def _():
            pltpu.sync_copy(acc.at[pl.ds(NS * rows_pw, tail)],
                            out_hbm.at[c, pl.ds(NS * rows_pw, tail)])

    return k(dst3)


def _gather_rows(table, idx):
    rows_pw = 400
    nsteps = rows_pw // CH

    @functools.partial(
        pl.kernel,
        out_type=jax.ShapeDtypeStruct((N, D), jnp.float32),
        mesh=plsc.VectorSubcoreMesh(**_MESH),
        scratch_types=[
            pltpu.VMEM((CH,), jnp.int32),
            pltpu.VMEM((CH, D), jnp.float32),
            pltpu.SemaphoreType.DMA,
        ],
    )
    def k(table_hbm, idx_hbm, out_hbm, idx_v, rows_v, sem):
        c = lax.axis_index("c")
        s = lax.axis_index("s")
        wid = c * NS + s

        @pl.when(wid < N // rows_pw)
        def _():
            def step(j, _):
                base = wid * rows_pw + j * CH
                pltpu.sync_copy(idx_hbm.at[pl.ds(base, CH)], idx_v)
                pltpu.async_copy(table_hbm.at[idx_v], rows_v, sem).wait()
                pltpu.sync_copy(rows_v, out_hbm.at[pl.ds(base, CH)])
                return 0

            lax.fori_loop(0, nsteps, step, 0)

    return k(table, idx)


BR = 1000


def _xw_body(x_ref, w_ref, o_ref):
    o_ref[...] = jnp.dot(x_ref[0], w_ref[0],
                         preferred_element_type=jnp.float32)[None]


def _tc_xw(xs, W):
    return pl.pallas_call(
        _xw_body,
        grid=(2, N // BR),
        in_specs=[
            pl.BlockSpec((1, BR, D), lambda a, b: (a, b, 0)),
            pl.BlockSpec((1, D, D), lambda a, b: (a, 0, 0)),
        ],
        out_specs=pl.BlockSpec((1, BR, D), lambda a, b: (a, b, 0)),
        out_shape=jax.ShapeDtypeStruct((2, N, D), jnp.float32),
    )(xs, W)


def _scale_body(raw_ref, cnt_ref, o_ref):
    cvec = cnt_ref[0][:, 0:1]
    inv = jnp.where(cvec > 0, 1.0 / cvec, 0.0)
    o_ref[...] = (raw_ref[0] * inv)[None]


def _tc_scale(raw, counts4, i):
    return pl.pallas_call(
        _scale_body,
        grid=(2, N // BR),
        in_specs=[
            pl.BlockSpec((1, BR, D), lambda a, b: (a, b, 0)),
            pl.BlockSpec((1, BR, D), lambda a, b: (2 * i + 1 - a, b, 0)),
        ],
        out_specs=pl.BlockSpec((1, BR, D), lambda a, b: (a, b, 0)),
        out_shape=jax.ShapeDtypeStruct((2, N, D), jnp.float32),
    )(raw, counts4)


def _post_body(raw_ref, cnt_ref, st_ref, nz_ref, bhg_ref, wm_ref, bm_ref,
               ws_ref, bs_ref, g_ref, b_ref, o_ref):
    cvec = cnt_ref[0][:, 0:1]
    dinv = jnp.where(cvec > 0, 1.0 / cvec, 0.0)
    conv = raw_ref[0] * dinv + bhg_ref[0]
    h = jnp.maximum(conv, 0.0)
    mean = jnp.dot(h, wm_ref[0], preferred_element_type=jnp.float32) \
        + bm_ref[0]
    logstd = jnp.dot(h, ws_ref[0], preferred_element_type=jnp.float32) \
        + bs_ref[0]
    enc = nz_ref[0] * jnp.exp(logstd) + mean
    y = enc + st_ref[0]
    mu = jnp.mean(y, axis=-1, keepdims=True)
    var = jnp.mean((y - mu) ** 2, axis=-1, keepdims=True)
    o_ref[...] = ((y - mu) / jnp.sqrt(var + EPS) * g_ref[0] + b_ref[0])[None]


def _tc_post(raw, counts4, i, state, noise, bhg, Wm, bm, Ws, bs, g, b):
    row = lambda a, bb: (a, bb, 0)
    par = lambda a, bb: (a, 0, 0)
    return pl.pallas_call(
        _post_body,
        grid=(2, N // BR),
        in_specs=[
            pl.BlockSpec((1, BR, D), row),
            pl.BlockSpec((1, BR, D), lambda a, bb: (2 * i + a, bb, 0)),
            pl.BlockSpec((1, BR, D), row),
            pl.BlockSpec((1, BR, D), row),
            pl.BlockSpec((1, 1, D), par),
            pl.BlockSpec((1, D, D), par),
            pl.BlockSpec((1, 1, D), par),
            pl.BlockSpec((1, D, D), par),
            pl.BlockSpec((1, 1, D), par),
            pl.BlockSpec((1, 1, D), par),
            pl.BlockSpec((1, 1, D), par),
        ],
        out_specs=pl.BlockSpec((1, BR, D), row),
        out_shape=jax.ShapeDtypeStruct((2, N, D), jnp.float32),
    )(raw, counts4, state, noise, bhg, Wm, bm, Ws, bs, g, b)


def kernel(x, edge_index_0, edge_index_1, root_users, params):
    graphs = [edge_index_0.astype(jnp.int32), edge_index_1.astype(jnp.int32)]
    root = root_users.astype(jnp.int32)

    key = jax.random.key(42)
    noises = [jax.random.normal(jax.random.fold_in(key, t), (N, D),
                                jnp.float32) for t in range(2 * NI)]

    cnt_r0 = _counts(jnp.concatenate([graphs[0][0], graphs[1][0]]))
    cnt_r1 = _counts(jnp.concatenate([graphs[0][1], graphs[1][1]]))
    counts4 = jnp.stack([cnt_r0[0], cnt_r1[0], cnt_r0[1], cnt_r1[1]])

    cas0 = _gather_rows(x, root)
    state = jnp.stack([x, cas0])

    pu, pc = params['user_vae'], params['cas_vae']
    Whg = jnp.stack([pu['Whg'], pc['Whg']])
    Wm = jnp.stack([pu['Wm'], pc['Wm']])
    Ws = jnp.stack([pu['Ws'], pc['Ws']])
    bhg = jnp.stack([pu['bhg'], pc['bhg']])[:, None, :]
    bm = jnp.stack([pu['bm'], pc['bm']])[:, None, :]
    bs = jnp.stack([pu['bs'], pc['bs']])[:, None, :]

    outs = []
    for i in range(NI):
        g0, g1 = graphs[i][0], graphs[i][1]
        src_p1 = jnp.concatenate([g0, g1 + N])
        dst_p1 = jnp.concatenate([g1, g0])
        src_p2 = jnp.concatenate([g1, g0 + N])
        dst_p2 = jnp.concatenate([g0, g1])

        xw = _tc_xw(state, Whg)
        raw1 = _seg_sum(xw.reshape(2 * N, D), src_p1, dst_p1)
        efeat = _tc_scale(raw1, counts4, i)
        raw2 = _seg_sum(efeat.reshape(2 * N, D), src_p2, dst_p2)

        noise_i = jnp.stack([noises[2 * i], noises[2 * i + 1]])
        lng = jnp.stack([params['un_g'][i], params['cn_g'][i]])[:, None, :]
        lnb = jnp.stack([params['un_b'][i], params['cn_b'][i]])[:, None, :]
        state = _tc_post(raw2, counts4, i, state, noise_i, bhg, Wm, bm, Ws,
                         bs, lng, lnb)
        outs.append(state[0])
        outs.append(state[1])
    return tuple(outs)
